# baseline (speedup 1.0000x reference)
import functools

import jax
import jax.numpy as jnp
from jax import lax
from jax.experimental import pallas as pl
from jax.experimental.pallas import tpu as pltpu
from jax.experimental.pallas import tpu_sc as plsc

N = 10000
E = 320000
D = 128
H = 64
WID = 128

NC = 2
NS = 16
NW = NC * NS
EPW = E // NW
CH = 80
NCHUNK = EPW // CH
NBUF = 5
CHS = 40
NCHUNKS = EPW // CHS

BE = 6400
NSLICE = 3


def _silu(x):
    return x * jax.nn.sigmoid(x)


def _prep_body(f_ref, c64_ref, w1a_ref, w1b_ref, b1_ref, rr_ref, rc_ref):
    f = f_ref[...]
    rr_ref[:, :H] = (jnp.dot(f, w1a_ref[...], preferred_element_type=jnp.float32)
                     + b1_ref[...])
    rr_ref[:, H:] = c64_ref[...]
    rc_ref[:, :H] = jnp.dot(f, w1b_ref[...], preferred_element_type=jnp.float32)
    rc_ref[:, H:] = -c64_ref[...]


def _prep(features, coords64, w1a, w1b, b1):
    return pl.pallas_call(
        _prep_body,
        out_shape=[
            jax.ShapeDtypeStruct((N, WID), jnp.float32),
            jax.ShapeDtypeStruct((N, WID), jnp.float32),
        ],
    )(features, coords64, w1a, w1b, b1)


def _make_gather(epw, nchunk, ch, nbuf):
    es = epw * NW

    def _gather_body(rr_hbm, rc_hbm, row3_hbm, col3_hbm, z_hbm, *scr):
        idxr_v, idxc_v = scr[0], scr[1]
        bufs = scr[2:2 + nbuf]
        sems = scr[2 + nbuf:2 + 2 * nbuf]
        wsem = scr[2 + 2 * nbuf]
        wid = lax.axis_index("s") * NC + lax.axis_index("c")
        base = wid * epw

        pltpu.sync_copy(row3_hbm.at[wid], idxr_v)
        pltpu.sync_copy(col3_hbm.at[wid], idxc_v)

        def body(g, carry):
            i0 = g * nbuf
            gds = []
            for b in range(nbuf):
                gd = pltpu.make_async_copy(
                    rr_hbm.at[idxr_v.at[i0 + b]], bufs[b], sems[b])
                gd.start()
                gds.append(gd)
            ads = []
            for b in range(nbuf):
                gds[b].wait()
                ad = pltpu.make_async_copy(
                    rc_hbm.at[idxc_v.at[i0 + b]], bufs[b], sems[b])
                ad.start(add=True)
                ads.append(ad)
            wds = []
            for b in range(nbuf):
                ads[b].wait()
                wd = pltpu.make_async_copy(
                    bufs[b], z_hbm.at[pl.ds(base + (i0 + b) * ch, ch)], wsem)
                wd.start()
                wds.append(wd)
            for b in range(nbuf):
                wds[b].wait()
            return carry

        lax.fori_loop(0, nchunk // nbuf, body, 0)

    mesh = plsc.VectorSubcoreMesh(
        core_axis_name="c", subcore_axis_name="s", num_cores=NC, num_subcores=NS)
    return functools.partial(
        pl.kernel,
        out_type=jax.ShapeDtypeStruct((es, WID), jnp.float32),
        mesh=mesh,
        scratch_types=[
            pltpu.VMEM((nchunk, ch), jnp.int32),
            pltpu.VMEM((nchunk, ch), jnp.int32),
        ] + [pltpu.VMEM((ch, WID), jnp.float32)] * nbuf
          + [pltpu.SemaphoreType.DMA] * (nbuf + 1),
    )(_gather_body)


def _edge_body(z_ref, w1c_ref, w2_ref, b2_ref,
               w3_ref, b3_ref, w4_ref, b4_ref, out_ref):
    zs = z_ref[:, :H]
    diff16 = z_ref[:, H:]
    dist = jnp.sum(diff16 * diff16, axis=1, keepdims=True)
    m = _silu(zs + dist * w1c_ref[...])
    msg = _silu(jnp.dot(m, w2_ref[...], preferred_element_type=jnp.float32)
                + b2_ref[...])
    c3 = _silu(jnp.dot(msg, w3_ref[...], preferred_element_type=jnp.float32)
               + b3_ref[...])
    cw = jnp.sum(c3 * w4_ref[...], axis=1, keepdims=True) + b4_ref[...]
    out_ref[:, :H] = msg
    out_ref[:, H:] = cw * diff16


def _edge_mlp(z, w1c, w2, b2, w3, b3, w4r, b4):
    nblk = z.shape[0] // BE
    blk = lambda i: (i, 0)
    fixed = lambda i: (0, 0)
    return pl.pallas_call(
        _edge_body,
        grid=(nblk,),
        in_specs=[
            pl.BlockSpec((BE, WID), blk),
            pl.BlockSpec((1, H), fixed),
            pl.BlockSpec((H, H), fixed),
            pl.BlockSpec((1, H), fixed),
            pl.BlockSpec((H, H), fixed),
            pl.BlockSpec((1, H), fixed),
            pl.BlockSpec((1, H), fixed),
            pl.BlockSpec((1, 1), fixed),
        ],
        out_specs=pl.BlockSpec((BE, WID), blk),
        out_shape=jax.ShapeDtypeStruct((z.shape[0], WID), jnp.float32),
    )(z, w1c, w2, b2, w3, b3, w4r, b4)


def _make_scatter(epw, nchunks, chs):
    es = epw * NW

    def _scatter_body(u_hbm, row2s_hbm, zeros_hbm, out_hbm,
                      i0, i1, i2, i3, i4, b0, b1, b2, b3, b4,
                      t0, t1, t2, t3, t4, s0, s1, s2, s3, s4, acc_sh):
        cid = lax.axis_index("c")
        sid = lax.axis_index("s")
        wid = sid * NC + cid
        base = wid * epw
        ibufs = (i0, i1, i2, i3, i4)
        bufs = (b0, b1, b2, b3, b4)
        isems = (t0, t1, t2, t3, t4)
        sems = (s0, s1, s2, s3, s4)

        @pl.when(sid == 0)
        def _():
            pltpu.sync_copy(zeros_hbm, acc_sh)

        plsc.subcore_barrier()

        def body(g, carry):
            i0_ = g * NBUF
            ids = []
            pds = []
            for b in range(NBUF):
                idd = pltpu.make_async_copy(
                    row2s_hbm.at[wid * nchunks + i0_ + b], ibufs[b], isems[b])
                idd.start()
                ids.append(idd)
                pd = pltpu.make_async_copy(
                    u_hbm.at[pl.ds(base + (i0_ + b) * chs, chs)], bufs[b], sems[b])
                pd.start()
                pds.append(pd)
            sds = []
            for b in range(NBUF):
                ids[b].wait()
                pds[b].wait()
                sd = pltpu.make_async_copy(bufs[b], acc_sh.at[ibufs[b]], sems[b])
                sd.start(add=True)
                sds.append(sd)
            for b in range(NBUF):
                sds[b].wait()
            return carry

        lax.fori_loop(0, nchunks // NBUF, body, 0)
        plsc.subcore_barrier()

        @pl.when(sid == 0)
        def _():
            pltpu.sync_copy(acc_sh, out_hbm.at[cid])

    mesh = plsc.VectorSubcoreMesh(
        core_axis_name="c", subcore_axis_name="s", num_cores=NC, num_subcores=NS)
    return functools.partial(
        pl.kernel,
        out_type=jax.ShapeDtypeStruct((NC, N, WID), jnp.float32),
        mesh=mesh,
        scratch_types=[pltpu.VMEM((chs,), jnp.int32)] * NBUF
          + [pltpu.VMEM((chs, WID), jnp.float32)] * NBUF
          + [pltpu.SemaphoreType.DMA] * (2 * NBUF)
          + [pltpu.VMEM_SHARED((N, WID), jnp.float32)],
    )(_scatter_body)


def _final_body(*refs):
    f_ref, c64_ref = refs[0], refs[1]
    scat_refs = refs[2:2 + NSLICE]
    w5a_ref, w5b_ref, b5_ref, nf_ref, nc64_ref = refs[2 + NSLICE:]
    f = f_ref[...]
    agg = scat_refs[0][0, :, :H] + scat_refs[0][1, :, :H]
    cupd = scat_refs[0][0, :, H:] + scat_refs[0][1, :, H:]
    for s_ref in scat_refs[1:]:
        agg = agg + s_ref[0, :, :H] + s_ref[1, :, :H]
        cupd = cupd + s_ref[0, :, H:] + s_ref[1, :, H:]
    pre = (jnp.dot(f, w5a_ref[...], preferred_element_type=jnp.float32)
           + jnp.dot(agg, w5b_ref[...], preferred_element_type=jnp.float32)
           + b5_ref[...])
    nf_ref[...] = _silu(pre)
    nc64_ref[...] = c64_ref[...] + cupd


def _final(features, coords64, scats, w5a, w5b, b5):
    bn = 2000
    nblk = N // bn
    blk2 = lambda i: (i, 0)
    blk3 = lambda i: (0, i, 0)
    fixed = lambda i: (0, 0)
    return pl.pallas_call(
        _final_body,
        grid=(nblk,),
        in_specs=[
            pl.BlockSpec((bn, D), blk2),
            pl.BlockSpec((bn, H), blk2),
        ] + [pl.BlockSpec((NC, bn, WID), blk3)] * NSLICE + [
            pl.BlockSpec((D, D), fixed),
            pl.BlockSpec((H, D), fixed),
            pl.BlockSpec((1, D), fixed),
        ],
        out_specs=[
            pl.BlockSpec((bn, D), blk2),
            pl.BlockSpec((bn, H), blk2),
        ],
        out_shape=[
            jax.ShapeDtypeStruct((N, D), jnp.float32),
            jax.ShapeDtypeStruct((N, H), jnp.float32),
        ],
    )(features, coords64, *scats, w5a, w5b, b5)


def kernel(features, coords, edge_index, W1, b1, W2, b2, W3, b3, W4, b4, W5, b5):
    row = edge_index[0].astype(jnp.int32)
    col = edge_index[1].astype(jnp.int32)
    coords64 = jnp.pad(coords, ((0, 0), (0, 61)))

    w1a = W1[:D]
    w1b = W1[D:2 * D]
    w1c = W1[2 * D].reshape(1, H)

    rr, rc = _prep(features, coords64, w1a, w1b, b1.reshape(1, H))

    row_g = row.reshape(NW, NCHUNK, CH)
    col_g = col.reshape(NW, NCHUNK, CH)
    row_s = row.reshape(NW, NCHUNKS, CHS)
    bounds_g = (0, 40, 80, NCHUNK)
    bounds_s = (0, 80, 160, NCHUNKS)
    zeros = jnp.zeros((N, WID), jnp.float32)

    scats = []
    for si in range(NSLICE):
        ncg = bounds_g[si + 1] - bounds_g[si]
        ncs = bounds_s[si + 1] - bounds_s[si]
        epw = ncg * CH
        gather = _make_gather(epw, ncg, CH, NBUF)
        scatter = _make_scatter(epw, ncs, CHS)
        r3 = row_g[:, bounds_g[si]:bounds_g[si + 1]]
        c3 = col_g[:, bounds_g[si]:bounds_g[si + 1]]
        r2s = row_s[:, bounds_s[si]:bounds_s[si + 1]].reshape(NW * ncs, CHS)
        z = gather(rr, rc, r3, c3)
        payload = _edge_mlp(
            z, w1c, W2, b2.reshape(1, H),
            W3, b3.reshape(1, H), W4.reshape(1, H), b4.reshape(1, 1))
        scats.append(scatter(payload, r2s, zeros))

    new_features, nc64 = _final(features, coords64, scats,
                                W5[:D], W5[D:], b5.reshape(1, D))
    return (new_features, nc64[:, :3])


# speedup vs baseline: 1.0147x; 1.0147x over previous
import functools

import jax
import jax.numpy as jnp
from jax import lax
from jax.experimental import pallas as pl
from jax.experimental.pallas import tpu as pltpu
from jax.experimental.pallas import tpu_sc as plsc

N = 10000
E = 320000
D = 128
H = 64
WID = 128

NC = 2
NS = 16
NW = NC * NS
EPW = E // NW
CH = 80
NCHUNK = EPW // CH
NBUF = 5
CHS = 40
NCHUNKS = EPW // CHS

BE = 3200
NSLICE = 3


def _silu(x):
    return x * jax.nn.sigmoid(x)


def _prep_body(f_ref, c64_ref, w1a_ref, w1b_ref, b1_ref, rr_ref, rc_ref):
    f = f_ref[...]
    rr_ref[:, :H] = (jnp.dot(f, w1a_ref[...], preferred_element_type=jnp.float32)
                     + b1_ref[...])
    rr_ref[:, H:] = c64_ref[...]
    rc_ref[:, :H] = jnp.dot(f, w1b_ref[...], preferred_element_type=jnp.float32)
    rc_ref[:, H:] = -c64_ref[...]


def _prep(features, coords64, w1a, w1b, b1):
    return pl.pallas_call(
        _prep_body,
        out_shape=[
            jax.ShapeDtypeStruct((N, WID), jnp.float32),
            jax.ShapeDtypeStruct((N, WID), jnp.float32),
        ],
    )(features, coords64, w1a, w1b, b1)


def _make_gather(epw, nchunk, ch, nbuf):
    es = epw * NW

    def _gather_body(rr_hbm, rc_hbm, row3_hbm, col3_hbm, z_hbm, *scr):
        idxr_v, idxc_v = scr[0], scr[1]
        bufs = scr[2:2 + nbuf]
        sems = scr[2 + nbuf:2 + 2 * nbuf]
        wsem = scr[2 + 2 * nbuf]
        wid = lax.axis_index("s") * NC + lax.axis_index("c")
        base = wid * epw

        pltpu.sync_copy(row3_hbm.at[wid], idxr_v)
        pltpu.sync_copy(col3_hbm.at[wid], idxc_v)

        def body(g, carry):
            i0 = g * nbuf
            gds = []
            for b in range(nbuf):
                gd = pltpu.make_async_copy(
                    rr_hbm.at[idxr_v.at[i0 + b]], bufs[b], sems[b])
                gd.start()
                gds.append(gd)
            ads = []
            for b in range(nbuf):
                gds[b].wait()
                ad = pltpu.make_async_copy(
                    rc_hbm.at[idxc_v.at[i0 + b]], bufs[b], sems[b])
                ad.start(add=True)
                ads.append(ad)
            wds = []
            for b in range(nbuf):
                ads[b].wait()
                wd = pltpu.make_async_copy(
                    bufs[b], z_hbm.at[pl.ds(base + (i0 + b) * ch, ch)], wsem)
                wd.start()
                wds.append(wd)
            for b in range(nbuf):
                wds[b].wait()
            return carry

        lax.fori_loop(0, nchunk // nbuf, body, 0)

    mesh = plsc.VectorSubcoreMesh(
        core_axis_name="c", subcore_axis_name="s", num_cores=NC, num_subcores=NS)
    return functools.partial(
        pl.kernel,
        out_type=jax.ShapeDtypeStruct((es, WID), jnp.float32),
        mesh=mesh,
        scratch_types=[
            pltpu.VMEM((nchunk, ch), jnp.int32),
            pltpu.VMEM((nchunk, ch), jnp.int32),
        ] + [pltpu.VMEM((ch, WID), jnp.float32)] * nbuf
          + [pltpu.SemaphoreType.DMA] * (nbuf + 1),
    )(_gather_body)


def _edge_body(z_ref, w1c_ref, w2_ref, b2_ref,
               w3_ref, b3_ref, w4_ref, b4_ref, out_ref):
    zs = z_ref[:, :H]
    diff16 = z_ref[:, H:]
    dist = jnp.sum(diff16 * diff16, axis=1, keepdims=True)
    m = _silu(zs + dist * w1c_ref[...])
    msg = _silu(jnp.dot(m, w2_ref[...], preferred_element_type=jnp.float32)
                + b2_ref[...])
    c3 = _silu(jnp.dot(msg, w3_ref[...], preferred_element_type=jnp.float32)
               + b3_ref[...])
    cw = jnp.sum(c3 * w4_ref[...], axis=1, keepdims=True) + b4_ref[...]
    out_ref[:, :H] = msg
    out_ref[:, H:] = cw * diff16


def _edge_mlp(z, w1c, w2, b2, w3, b3, w4r, b4):
    nblk = z.shape[0] // BE
    blk = lambda i: (i, 0)
    fixed = lambda i: (0, 0)
    return pl.pallas_call(
        _edge_body,
        grid=(nblk,),
        in_specs=[
            pl.BlockSpec((BE, WID), blk),
            pl.BlockSpec((1, H), fixed),
            pl.BlockSpec((H, H), fixed),
            pl.BlockSpec((1, H), fixed),
            pl.BlockSpec((H, H), fixed),
            pl.BlockSpec((1, H), fixed),
            pl.BlockSpec((1, H), fixed),
            pl.BlockSpec((1, 1), fixed),
        ],
        out_specs=pl.BlockSpec((BE, WID), blk),
        out_shape=jax.ShapeDtypeStruct((z.shape[0], WID), jnp.float32),
    )(z, w1c, w2, b2, w3, b3, w4r, b4)


def _make_scatter(epw, nchunks, chs):
    es = epw * NW

    def _scatter_body(u_hbm, row2s_hbm, zeros_hbm, out_hbm,
                      i0, i1, i2, i3, i4, b0, b1, b2, b3, b4,
                      t0, t1, t2, t3, t4, s0, s1, s2, s3, s4, acc_sh):
        cid = lax.axis_index("c")
        sid = lax.axis_index("s")
        wid = sid * NC + cid
        base = wid * epw
        ibufs = (i0, i1, i2, i3, i4)
        bufs = (b0, b1, b2, b3, b4)
        isems = (t0, t1, t2, t3, t4)
        sems = (s0, s1, s2, s3, s4)

        @pl.when(sid == 0)
        def _():
            pltpu.sync_copy(zeros_hbm, acc_sh)

        plsc.subcore_barrier()

        def body(g, carry):
            i0_ = g * NBUF
            ids = []
            pds = []
            for b in range(NBUF):
                idd = pltpu.make_async_copy(
                    row2s_hbm.at[wid * nchunks + i0_ + b], ibufs[b], isems[b])
                idd.start()
                ids.append(idd)
                pd = pltpu.make_async_copy(
                    u_hbm.at[pl.ds(base + (i0_ + b) * chs, chs)], bufs[b], sems[b])
                pd.start()
                pds.append(pd)
            sds = []
            for b in range(NBUF):
                ids[b].wait()
                pds[b].wait()
                sd = pltpu.make_async_copy(bufs[b], acc_sh.at[ibufs[b]], sems[b])
                sd.start(add=True)
                sds.append(sd)
            for b in range(NBUF):
                sds[b].wait()
            return carry

        lax.fori_loop(0, nchunks // NBUF, body, 0)
        plsc.subcore_barrier()

        @pl.when(sid == 0)
        def _():
            pltpu.sync_copy(acc_sh, out_hbm.at[cid])

    mesh = plsc.VectorSubcoreMesh(
        core_axis_name="c", subcore_axis_name="s", num_cores=NC, num_subcores=NS)
    return functools.partial(
        pl.kernel,
        out_type=jax.ShapeDtypeStruct((NC, N, WID), jnp.float32),
        mesh=mesh,
        scratch_types=[pltpu.VMEM((chs,), jnp.int32)] * NBUF
          + [pltpu.VMEM((chs, WID), jnp.float32)] * NBUF
          + [pltpu.SemaphoreType.DMA] * (2 * NBUF)
          + [pltpu.VMEM_SHARED((N, WID), jnp.float32)],
    )(_scatter_body)


def _final_body(*refs):
    f_ref, c64_ref = refs[0], refs[1]
    scat_refs = refs[2:2 + NSLICE]
    w5a_ref, w5b_ref, b5_ref, nf_ref, nc64_ref = refs[2 + NSLICE:]
    f = f_ref[...]
    agg = scat_refs[0][0, :, :H] + scat_refs[0][1, :, :H]
    cupd = scat_refs[0][0, :, H:] + scat_refs[0][1, :, H:]
    for s_ref in scat_refs[1:]:
        agg = agg + s_ref[0, :, :H] + s_ref[1, :, :H]
        cupd = cupd + s_ref[0, :, H:] + s_ref[1, :, H:]
    pre = (jnp.dot(f, w5a_ref[...], preferred_element_type=jnp.float32)
           + jnp.dot(agg, w5b_ref[...], preferred_element_type=jnp.float32)
           + b5_ref[...])
    nf_ref[...] = _silu(pre)
    nc64_ref[...] = c64_ref[...] + cupd


def _final(features, coords64, scats, w5a, w5b, b5):
    bn = 2000
    nblk = N // bn
    blk2 = lambda i: (i, 0)
    blk3 = lambda i: (0, i, 0)
    fixed = lambda i: (0, 0)
    return pl.pallas_call(
        _final_body,
        grid=(nblk,),
        in_specs=[
            pl.BlockSpec((bn, D), blk2),
            pl.BlockSpec((bn, H), blk2),
        ] + [pl.BlockSpec((NC, bn, WID), blk3)] * NSLICE + [
            pl.BlockSpec((D, D), fixed),
            pl.BlockSpec((H, D), fixed),
            pl.BlockSpec((1, D), fixed),
        ],
        out_specs=[
            pl.BlockSpec((bn, D), blk2),
            pl.BlockSpec((bn, H), blk2),
        ],
        out_shape=[
            jax.ShapeDtypeStruct((N, D), jnp.float32),
            jax.ShapeDtypeStruct((N, H), jnp.float32),
        ],
    )(features, coords64, *scats, w5a, w5b, b5)


def kernel(features, coords, edge_index, W1, b1, W2, b2, W3, b3, W4, b4, W5, b5):
    row = edge_index[0].astype(jnp.int32)
    col = edge_index[1].astype(jnp.int32)
    coords64 = jnp.pad(coords, ((0, 0), (0, 61)))

    w1a = W1[:D]
    w1b = W1[D:2 * D]
    w1c = W1[2 * D].reshape(1, H)

    rr, rc = _prep(features, coords64, w1a, w1b, b1.reshape(1, H))

    row_g = row.reshape(NW, NCHUNK, CH)
    col_g = col.reshape(NW, NCHUNK, CH)
    row_s = row.reshape(NW, NCHUNKS, CHS)
    bounds_g = (0, 40, 80, NCHUNK)
    bounds_s = (0, 80, 160, NCHUNKS)
    zeros = jnp.zeros((N, WID), jnp.float32)

    scats = []
    for si in range(NSLICE):
        ncg = bounds_g[si + 1] - bounds_g[si]
        ncs = bounds_s[si + 1] - bounds_s[si]
        epw = ncg * CH
        gather = _make_gather(epw, ncg, CH, NBUF)
        scatter = _make_scatter(epw, ncs, CHS)
        r3 = row_g[:, bounds_g[si]:bounds_g[si + 1]]
        c3 = col_g[:, bounds_g[si]:bounds_g[si + 1]]
        r2s = row_s[:, bounds_s[si]:bounds_s[si + 1]].reshape(NW * ncs, CHS)
        z = gather(rr, rc, r3, c3)
        payload = _edge_mlp(
            z, w1c, W2, b2.reshape(1, H),
            W3, b3.reshape(1, H), W4.reshape(1, H), b4.reshape(1, 1))
        scats.append(scatter(payload, r2s, zeros))

    new_features, nc64 = _final(features, coords64, scats,
                                W5[:D], W5[D:], b5.reshape(1, D))
    return (new_features, nc64[:, :3])
